# baseline (speedup 1.0000x reference)
import functools

import jax
import jax.numpy as jnp
from jax import lax
from jax.experimental import pallas as pl
from jax.experimental.pallas import tpu as pltpu
from jax.experimental.pallas import tpu_sc as plsc

N = 10000
D = 128
H = 128
E = 320000

NC = 2
NS = 16
NT = NC * NS

NP = 10240
NB = NP // 128
EPB = 64
CHB = 16
NCH = 10
EPT = NCH * CHB * EPB
KB = EPT // EPB
EP = NT * EPT
RPT = NP // NS

_mesh = plsc.VectorSubcoreMesh(core_axis_name="c", subcore_axis_name="s")
_sc_params = pltpu.CompilerParams(needs_layout_passes=False)


@functools.partial(
    pl.kernel,
    out_type=jax.ShapeDtypeStruct((NT, NP), jnp.float32),
    mesh=_mesh,
    compiler_params=_sc_params,
    scratch_types=[
        pltpu.VMEM((EPT,), jnp.int32),
        pltpu.VMEM((NP,), jnp.float32),
    ],
)
def _deg_call(dst_hbm, degp_hbm, dst_v, deg_v):
    c = lax.axis_index("c")
    s = lax.axis_index("s")
    wid = c * NS + s

    def _zero(i, _):
        deg_v[pl.ds(i * 16, 16)] = jnp.zeros((16,), jnp.float32)
        return _

    lax.fori_loop(0, NP // 16, _zero, None)
    pltpu.sync_copy(dst_hbm.at[wid], dst_v)
    ones = jnp.ones((16,), jnp.float32)

    def _count(i, _):
        idx = dst_v[pl.ds(i * 16, 16)]
        plsc.addupdate_scatter(deg_v, [idx], ones)
        return _

    lax.fori_loop(0, EPT // 16, _count, None)
    pltpu.sync_copy(deg_v, degp_hbm.at[wid])


@functools.partial(
    pl.kernel,
    out_type=jax.ShapeDtypeStruct((NC, NP, 128), jnp.float32),
    mesh=_mesh,
    compiler_params=_sc_params,
    scratch_types=[
        pltpu.VMEM((2, CHB, EPB), jnp.int32),
        pltpu.VMEM((2, CHB, EPB), jnp.int32),
        pltpu.VMEM((EPB, 128), jnp.float32),
        pltpu.VMEM((EPB, 128), jnp.float32),
        pltpu.VMEM_SHARED((NP, 128), jnp.float32),
        pltpu.SemaphoreType.DMA,
        pltpu.SemaphoreType.DMA,
        pltpu.SemaphoreType.DMA,
    ],
)
def _edge_call(hs_hbm, src_hbm, dst_hbm, zer_hbm, out_hbm,
               src_c, dst_c, rows_a, rows_b, acc_sh, sem_a, sem_b, sem_i):
    c = lax.axis_index("c")
    s = lax.axis_index("s")
    wid = c * NS + s

    pltpu.sync_copy(zer_hbm, acc_sh.at[pl.ds(s * RPT, RPT)])
    pltpu.sync_copy(src_hbm.at[wid, 0], src_c.at[0])
    pltpu.sync_copy(dst_hbm.at[wid, 0], dst_c.at[0])
    plsc.subcore_barrier()

    def _chunk(k, q):
        @pl.when(k + 1 < NCH)
        def _():
            pltpu.async_copy(src_hbm.at[wid, k + 1], src_c.at[1 - q], sem_i)
            pltpu.async_copy(dst_hbm.at[wid, k + 1], dst_c.at[1 - q], sem_i)

        pltpu.async_copy(hs_hbm.at[src_c.at[q, 0]], rows_a, sem_a)

        def _pair(p, _):
            j = 2 * p
            pltpu.async_copy(hs_hbm.at[src_c.at[q, j + 1]], rows_b, sem_b)
            pltpu.make_async_copy(hs_hbm.at[src_c.at[q, j]], rows_a,
                                  sem_a).wait()
            pltpu.sync_copy(rows_a, acc_sh.at[dst_c.at[q, j]], add=True)

            @pl.when(j + 2 < CHB)
            def _():
                pltpu.async_copy(hs_hbm.at[src_c.at[q, j + 2]], rows_a, sem_a)

            pltpu.make_async_copy(hs_hbm.at[src_c.at[q, j + 1]], rows_b,
                                  sem_b).wait()
            pltpu.sync_copy(rows_b, acc_sh.at[dst_c.at[q, j + 1]], add=True)
            return _

        lax.fori_loop(0, CHB // 2, _pair, None)

        @pl.when(k + 1 < NCH)
        def _():
            pltpu.make_async_copy(src_hbm.at[wid, 0], src_c.at[1 - q],
                                  sem_i).wait()
            pltpu.make_async_copy(dst_hbm.at[wid, 0], dst_c.at[1 - q],
                                  sem_i).wait()

    def _dchunk(d, _):
        _chunk(2 * d, 0)
        _chunk(2 * d + 1, 1)
        return _

    lax.fori_loop(0, NCH // 2, _dchunk, None)
    plsc.subcore_barrier()
    pltpu.sync_copy(acc_sh.at[pl.ds(s * RPT, RPT)],
                    out_hbm.at[c, pl.ds(s * RPT, RPT)])


def _dinv_body(degp_ref, dinv_ref):
    deg = jnp.sum(degp_ref[...], axis=0) + 1.0
    r = lax.rsqrt(deg)
    row = lax.broadcasted_iota(jnp.int32, (NB, 128), 0)
    col = lax.broadcasted_iota(jnp.int32, (NB, 128), 1)
    dinv_ref[...] = jnp.where(row * 128 + col < N, r, 0.0)


def _mm_body(x_ref, w_ref, dv_ref, o_ref):
    h = jnp.dot(x_ref[...], w_ref[...], preferred_element_type=jnp.float32)
    o_ref[...] = h * dv_ref[...]


def _pre_body(p_ref, hs_ref, dv_ref, pre_ref, sums_ref):
    t = (p_ref[0] + p_ref[1] + hs_ref[...]) * dv_ref[...]
    pre_ref[...] = t

    @pl.when(pl.program_id(0) == 0)
    def _():
        sums_ref[...] = jnp.zeros_like(sums_ref)

    sums_ref[0:1, :] += jnp.sum(t, axis=0, keepdims=True)
    sums_ref[1:2, :] += jnp.sum(t * t, axis=0, keepdims=True)


def _bn_stats(sums_ref, g_ref, be_ref):
    mu = sums_ref[0:1, :] / N
    var = sums_ref[1:2, :] / N - mu * mu
    scale = g_ref[...] * lax.rsqrt(var + 1e-5)
    shift = be_ref[...] - mu * scale
    return scale, shift


def _bn_mm_body(pre_ref, sums_ref, g_ref, be_ref, w_ref, dv_ref, o_ref):
    scale, shift = _bn_stats(sums_ref, g_ref, be_ref)
    y = jnp.maximum(pre_ref[...] * scale + shift, 0.0)
    h = jnp.dot(y, w_ref[...], preferred_element_type=jnp.float32)
    o_ref[...] = h * dv_ref[...]


def _head_body(pre_ref, sums_ref, g_ref, be_ref, w_ref, fb_ref, o_ref):
    scale, shift = _bn_stats(sums_ref, g_ref, be_ref)
    y = jnp.maximum(pre_ref[...] * scale + shift, 0.0)
    t = jnp.dot(y, w_ref[...], preferred_element_type=jnp.float32) + fb_ref[...]
    o_ref[...] = jax.nn.sigmoid(t)


_f32 = jnp.float32
_blk = pl.BlockSpec((128, 128), lambda i: (i, 0))
_col = pl.BlockSpec((128, 1), lambda i: (i, 0))
_full = pl.BlockSpec((128, 128), lambda i: (0, 0))
_vec = pl.BlockSpec((1, 128), lambda i: (0, 0))
_sums = pl.BlockSpec((8, 128), lambda i: (0, 0))


def _dinv_call(degp):
    return pl.pallas_call(
        _dinv_body,
        out_shape=jax.ShapeDtypeStruct((NB, 128), _f32),
    )(degp.reshape(NT, NB, 128))


def _mm_call(x, w, dv):
    return pl.pallas_call(
        _mm_body, grid=(NB,),
        in_specs=[_blk, _full, _col],
        out_specs=_blk,
        out_shape=jax.ShapeDtypeStruct((NP, 128), _f32),
    )(x, w, dv)


def _pre_call(p, hs, dv):
    return pl.pallas_call(
        _pre_body, grid=(NB,),
        in_specs=[pl.BlockSpec((NC, 128, 128), lambda i: (0, i, 0)), _blk, _col],
        out_specs=[_blk, _sums],
        out_shape=[jax.ShapeDtypeStruct((NP, 128), _f32),
                   jax.ShapeDtypeStruct((8, 128), _f32)],
    )(p, hs, dv)


def _bn_mm_call(pre, sums, g, be, w, dv):
    return pl.pallas_call(
        _bn_mm_body, grid=(NB,),
        in_specs=[_blk, _sums, _vec, _vec, _full, _col],
        out_specs=_blk,
        out_shape=jax.ShapeDtypeStruct((NP, 128), _f32),
    )(pre, sums, g.reshape(1, 128), be.reshape(1, 128), w, dv)


def _head_call(pre, sums, g, be, wp, fb):
    return pl.pallas_call(
        _head_body, grid=(NB,),
        in_specs=[_blk, _sums, _vec, _vec, _full, _vec],
        out_specs=_blk,
        out_shape=jax.ShapeDtypeStruct((NP, 128), _f32),
    )(pre, sums, g.reshape(1, 128), be.reshape(1, 128), wp, fb)


def kernel(x, edge_index, W1, b1, g1, be1, W2, b2, g2, be2, W3, b3, g3, be3,
           fcW, fcb):
    del b1, b2, b3
    src = edge_index[0]
    dst = edge_index[1]
    fill = jnp.full((EP - E,), N, jnp.int32)
    src_p = jnp.concatenate([src, fill])
    dst_p = jnp.concatenate([dst, fill])
    src3 = src_p.reshape(NT, NCH, CHB, EPB)
    dst3 = dst_p.reshape(NT, NCH, CHB, EPB)
    dst2 = dst_p.reshape(NT, EPT)
    x_pad = jnp.pad(x, ((0, NP - N), (0, 0)))
    zer = jnp.zeros((RPT, 128), _f32)
    fcWp = jnp.pad(fcW, ((0, 0), (0, 127)))
    fbv = jnp.broadcast_to(fcb, (128,)).reshape(1, 128)

    degp = _deg_call(dst2)
    dinv = _dinv_call(degp).reshape(NP, 1)

    hs = _mm_call(x_pad, W1, dinv)
    p1 = _edge_call(hs, src3, dst3, zer)
    pre1, sums1 = _pre_call(p1, hs, dinv)

    hs2 = _bn_mm_call(pre1, sums1, g1, be1, W2, dinv)
    p2 = _edge_call(hs2, src3, dst3, zer)
    pre2, sums2 = _pre_call(p2, hs2, dinv)

    hs3 = _bn_mm_call(pre2, sums2, g2, be2, W3, dinv)
    p3 = _edge_call(hs3, src3, dst3, zer)
    pre3, sums3 = _pre_call(p3, hs3, dinv)

    res = _head_call(pre3, sums3, g3, be3, fcWp, fbv)
    return res[:N, :1]


# speedup vs baseline: 2.3506x; 2.3506x over previous
import functools

import jax
import jax.numpy as jnp
from jax import lax
from jax.experimental import pallas as pl
from jax.experimental.pallas import tpu as pltpu
from jax.experimental.pallas import tpu_sc as plsc

N = 10000
D = 128
H = 128
E = 320000

NC = 2
NS = 16
NT = NC * NS

NP = 10240
NB = NP // 128
EPB = 128
KB = 160
EPT = KB * EPB
EP = NS * EPT
EPD = EP // NT
RPT = NP // NS

_mesh = plsc.VectorSubcoreMesh(core_axis_name="c", subcore_axis_name="s")
_sc_params = pltpu.CompilerParams(needs_layout_passes=False,
                                 use_tc_tiling_on_sc=False)


@functools.partial(
    pl.kernel,
    out_type=jax.ShapeDtypeStruct((NT, NP), jnp.float32),
    mesh=_mesh,
    compiler_params=_sc_params,
    scratch_types=[
        pltpu.VMEM((EPD,), jnp.int32),
        pltpu.VMEM((NP,), jnp.float32),
    ],
)
def _deg_call(dst_hbm, degp_hbm, dst_v, deg_v):
    c = lax.axis_index("c")
    s = lax.axis_index("s")
    wid = c * NS + s

    def _zero(i, _):
        deg_v[pl.ds(i * 16, 16)] = jnp.zeros((16,), jnp.float32)
        return _

    lax.fori_loop(0, NP // 16, _zero, None)
    pltpu.sync_copy(dst_hbm.at[wid], dst_v)
    ones = jnp.ones((16,), jnp.float32)

    def _count(i, _):
        idx = dst_v[pl.ds(i * 16, 16)]
        plsc.addupdate_scatter(deg_v, [idx], ones)
        return _

    lax.fori_loop(0, EPD // 16, _count, None)
    pltpu.sync_copy(deg_v, degp_hbm.at[wid])


@functools.partial(
    pl.kernel,
    out_type=jax.ShapeDtypeStruct((NC, NP, 64), jnp.float32),
    mesh=_mesh,
    compiler_params=_sc_params,
    scratch_types=[
        pltpu.VMEM((KB, EPB), jnp.int32),
        pltpu.VMEM((KB, EPB), jnp.int32),
        pltpu.VMEM((EPB, 64), jnp.float32),
        pltpu.VMEM((EPB, 64), jnp.float32),
        pltpu.VMEM((EPB, 64), jnp.float32),
        pltpu.VMEM_SHARED((NP, 64), jnp.float32),
        pltpu.SemaphoreType.DMA,
        pltpu.SemaphoreType.DMA,
        pltpu.SemaphoreType.DMA,
    ],
)
def _edge_call(hs_hbm, src_hbm, dst_hbm, zer_hbm, out_hbm,
               src_v, dst_v, rows_a, rows_b, rows_c, acc_sh,
               sem_a, sem_b, sem_c):
    c = lax.axis_index("c")
    s = lax.axis_index("s")

    pltpu.sync_copy(zer_hbm, acc_sh.at[pl.ds(s * RPT, RPT)])
    pltpu.sync_copy(src_hbm.at[c, s], src_v)
    pltpu.sync_copy(dst_hbm.at[s], dst_v)
    plsc.subcore_barrier()

    pltpu.async_copy(hs_hbm.at[src_v.at[0]], rows_a, sem_a)
    pltpu.async_copy(hs_hbm.at[src_v.at[1]], rows_b, sem_b)
    pltpu.async_copy(hs_hbm.at[src_v.at[2]], rows_c, sem_c)

    def _slot(j, buf, sem):
        pltpu.make_async_copy(hs_hbm.at[src_v.at[j]], buf, sem).wait()
        pltpu.sync_copy(buf, acc_sh.at[dst_v.at[j]], add=True)

        @pl.when(j + 3 < KB)
        def _():
            pltpu.async_copy(hs_hbm.at[src_v.at[j + 3]], buf, sem)

    def _triple(t, _):
        j = 3 * t
        _slot(j, rows_a, sem_a)
        _slot(j + 1, rows_b, sem_b)
        _slot(j + 2, rows_c, sem_c)
        return _

    lax.fori_loop(0, KB // 3, _triple, None)
    _slot(KB - 1, rows_a, sem_a)
    plsc.subcore_barrier()
    pltpu.sync_copy(acc_sh.at[pl.ds(s * RPT, RPT)],
                    out_hbm.at[c, pl.ds(s * RPT, RPT)])


def _dinv_body(degp_ref, dinv_ref):
    deg = jnp.sum(degp_ref[...], axis=0) + 1.0
    r = lax.rsqrt(deg)
    row = lax.broadcasted_iota(jnp.int32, (NB, 128), 0)
    col = lax.broadcasted_iota(jnp.int32, (NB, 128), 1)
    dinv_ref[...] = jnp.where(row * 128 + col < N, r, 0.0)


def _mm_body(x_ref, w_ref, dv_ref, o_ref):
    h = jnp.dot(x_ref[...], w_ref[...], preferred_element_type=jnp.float32)
    o_ref[...] = h * dv_ref[...]


def _pre_body(p_ref, hs_ref, dv_ref, pre_ref, sums_ref):
    p = jnp.concatenate([p_ref[0], p_ref[1]], axis=1)
    t = (p + hs_ref[...]) * dv_ref[...]
    pre_ref[...] = t

    @pl.when(pl.program_id(0) == 0)
    def _():
        sums_ref[...] = jnp.zeros_like(sums_ref)

    sums_ref[0:1, :] += jnp.sum(t, axis=0, keepdims=True)
    sums_ref[1:2, :] += jnp.sum(t * t, axis=0, keepdims=True)


def _bn_stats(sums_ref, g_ref, be_ref):
    mu = sums_ref[0:1, :] / N
    var = sums_ref[1:2, :] / N - mu * mu
    scale = g_ref[...] * lax.rsqrt(var + 1e-5)
    shift = be_ref[...] - mu * scale
    return scale, shift


def _bn_mm_body(pre_ref, sums_ref, g_ref, be_ref, w_ref, dv_ref, o_ref):
    scale, shift = _bn_stats(sums_ref, g_ref, be_ref)
    y = jnp.maximum(pre_ref[...] * scale + shift, 0.0)
    h = jnp.dot(y, w_ref[...], preferred_element_type=jnp.float32)
    o_ref[...] = h * dv_ref[...]


def _head_body(pre_ref, sums_ref, g_ref, be_ref, w_ref, fb_ref, o_ref):
    scale, shift = _bn_stats(sums_ref, g_ref, be_ref)
    y = jnp.maximum(pre_ref[...] * scale + shift, 0.0)
    t = jnp.dot(y, w_ref[...], preferred_element_type=jnp.float32) + fb_ref[...]
    o_ref[...] = jax.nn.sigmoid(t)


_f32 = jnp.float32
_blk = pl.BlockSpec((128, 128), lambda i: (i, 0))
_col = pl.BlockSpec((128, 1), lambda i: (i, 0))
_full = pl.BlockSpec((128, 128), lambda i: (0, 0))
_vec = pl.BlockSpec((1, 128), lambda i: (0, 0))
_sums = pl.BlockSpec((8, 128), lambda i: (0, 0))


def _dinv_call(degp):
    return pl.pallas_call(
        _dinv_body,
        out_shape=jax.ShapeDtypeStruct((NB, 128), _f32),
    )(degp.reshape(NT, NB, 128))


def _mm_call(x, w, dv):
    return pl.pallas_call(
        _mm_body, grid=(NB,),
        in_specs=[_blk, _full, _col],
        out_specs=_blk,
        out_shape=jax.ShapeDtypeStruct((NP, 128), _f32),
    )(x, w, dv)


def _pre_call(p, hs, dv):
    return pl.pallas_call(
        _pre_body, grid=(NB,),
        in_specs=[pl.BlockSpec((NC, 128, 64), lambda i: (0, i, 0)), _blk, _col],
        out_specs=[_blk, _sums],
        out_shape=[jax.ShapeDtypeStruct((NP, 128), _f32),
                   jax.ShapeDtypeStruct((8, 128), _f32)],
    )(p, hs, dv)


def _bn_mm_call(pre, sums, g, be, w, dv):
    return pl.pallas_call(
        _bn_mm_body, grid=(NB,),
        in_specs=[_blk, _sums, _vec, _vec, _full, _col],
        out_specs=_blk,
        out_shape=jax.ShapeDtypeStruct((NP, 128), _f32),
    )(pre, sums, g.reshape(1, 128), be.reshape(1, 128), w, dv)


def _head_call(pre, sums, g, be, wp, fb):
    return pl.pallas_call(
        _head_body, grid=(NB,),
        in_specs=[_blk, _sums, _vec, _vec, _full, _vec],
        out_specs=_blk,
        out_shape=jax.ShapeDtypeStruct((NP, 128), _f32),
    )(pre, sums, g.reshape(1, 128), be.reshape(1, 128), wp, fb)


def kernel(x, edge_index, W1, b1, g1, be1, W2, b2, g2, be2, W3, b3, g3, be3,
           fcW, fcb):
    del b1, b2, b3
    src = edge_index[0]
    dst = edge_index[1]
    fill = N + jnp.arange(EP - E, dtype=jnp.int32) % (NP - N)
    src_p = jnp.concatenate([src, fill])
    dst_p = jnp.concatenate([dst, fill])
    sb = (2 * src_p).reshape(NS, KB, EPB)
    src3 = jnp.stack([sb, sb + 1])
    dst3 = dst_p.reshape(NS, KB, EPB)
    dst2 = dst_p.reshape(NT, EPD)
    x_pad = jnp.pad(x, ((0, NP - N), (0, 0)))
    zer = jnp.zeros((RPT, 64), _f32)
    fcWp = jnp.pad(fcW, ((0, 0), (0, 127)))
    fbv = jnp.broadcast_to(fcb, (128,)).reshape(1, 128)

    degp = _deg_call(dst2)
    dinv = _dinv_call(degp).reshape(NP, 1)

    hs = _mm_call(x_pad, W1, dinv)
    p1 = _edge_call(hs.reshape(2 * NP, 64), src3, dst3, zer)
    pre1, sums1 = _pre_call(p1, hs, dinv)

    hs2 = _bn_mm_call(pre1, sums1, g1, be1, W2, dinv)
    p2 = _edge_call(hs2.reshape(2 * NP, 64), src3, dst3, zer)
    pre2, sums2 = _pre_call(p2, hs2, dinv)

    hs3 = _bn_mm_call(pre2, sums2, g2, be2, W3, dinv)
    p3 = _edge_call(hs3.reshape(2 * NP, 64), src3, dst3, zer)
    pre3, sums3 = _pre_call(p3, hs3, dinv)

    res = _head_call(pre3, sums3, g3, be3, fcWp, fbv)
    return res[:N, :1]


# speedup vs baseline: 2.3656x; 1.0064x over previous
import functools

import jax
import jax.numpy as jnp
from jax import lax
from jax.experimental import pallas as pl
from jax.experimental.pallas import tpu as pltpu
from jax.experimental.pallas import tpu_sc as plsc

N = 10000
D = 128
H = 128
E = 320000

NC = 2
NS = 16
NT = NC * NS

NP = 10240
NB = NP // 128
EPB = 128
KB = 160
EPT = KB * EPB
EP = NS * EPT
EPD = EP // NT
RPT = NP // NS

_mesh = plsc.VectorSubcoreMesh(core_axis_name="c", subcore_axis_name="s")
_sc_params = pltpu.CompilerParams(needs_layout_passes=False,
                                 use_tc_tiling_on_sc=False)


@functools.partial(
    pl.kernel,
    out_type=jax.ShapeDtypeStruct((NT, NP), jnp.float32),
    mesh=_mesh,
    compiler_params=_sc_params,
    scratch_types=[
        pltpu.VMEM((EPD,), jnp.int32),
        pltpu.VMEM((NP,), jnp.float32),
    ],
)
def _deg_call(dst_hbm, degp_hbm, dst_v, deg_v):
    c = lax.axis_index("c")
    s = lax.axis_index("s")
    wid = c * NS + s

    def _zero(i, _):
        deg_v[pl.ds(i * 16, 16)] = jnp.zeros((16,), jnp.float32)
        return _

    lax.fori_loop(0, NP // 16, _zero, None)
    pltpu.sync_copy(dst_hbm.at[wid], dst_v)
    ones = jnp.ones((16,), jnp.float32)

    def _count(i, _):
        idx = dst_v[pl.ds(i * 16, 16)]
        plsc.addupdate_scatter(deg_v, [idx], ones)
        return _

    lax.fori_loop(0, EPD // 16, _count, None)
    pltpu.sync_copy(deg_v, degp_hbm.at[wid])


@functools.partial(
    pl.kernel,
    out_type=jax.ShapeDtypeStruct((NC, NP, 64), jnp.float32),
    mesh=_mesh,
    compiler_params=_sc_params,
    scratch_types=[
        pltpu.VMEM((KB, EPB), jnp.int32),
        pltpu.VMEM((KB, EPB), jnp.int32),
        pltpu.VMEM((EPB, 64), jnp.float32),
        pltpu.VMEM((EPB, 64), jnp.float32),
        pltpu.VMEM((EPB, 64), jnp.float32),
        pltpu.VMEM_SHARED((NP, 64), jnp.float32),
        pltpu.SemaphoreType.DMA,
        pltpu.SemaphoreType.DMA,
        pltpu.SemaphoreType.DMA,
    ],
)
def _edge_call(hs_hbm, src_hbm, dst_hbm, zer_hbm, out_hbm,
               src_v, dst_v, rows_a, rows_b, rows_c, acc_sh,
               sem_a, sem_b, sem_c):
    c = lax.axis_index("c")
    s = lax.axis_index("s")

    pltpu.sync_copy(zer_hbm, acc_sh.at[pl.ds(s * RPT, RPT)])
    pltpu.sync_copy(src_hbm.at[c, s], src_v)
    pltpu.sync_copy(dst_hbm.at[s], dst_v)
    plsc.subcore_barrier()

    pltpu.async_copy(hs_hbm.at[src_v.at[0]], rows_a, sem_a)
    pltpu.async_copy(hs_hbm.at[src_v.at[1]], rows_b, sem_b)
    pltpu.async_copy(hs_hbm.at[src_v.at[2]], rows_c, sem_c)

    def _slot(j, buf, sem):
        pltpu.make_async_copy(hs_hbm.at[src_v.at[j]], buf, sem).wait()
        pltpu.sync_copy(buf, acc_sh.at[dst_v.at[j]], add=True)

        @pl.when(j + 3 < KB)
        def _():
            pltpu.async_copy(hs_hbm.at[src_v.at[j + 3]], buf, sem)

    def _triple(t, _):
        j = 3 * t
        _slot(j, rows_a, sem_a)
        _slot(j + 1, rows_b, sem_b)
        _slot(j + 2, rows_c, sem_c)
        return _

    lax.fori_loop(0, KB // 3, _triple, None)
    _slot(KB - 1, rows_a, sem_a)
    plsc.subcore_barrier()
    pltpu.sync_copy(acc_sh.at[pl.ds(s * RPT, RPT)],
                    out_hbm.at[c, pl.ds(s * RPT, RPT)])


def _dinv_body(degp_ref, dinv_ref):
    deg = jnp.sum(degp_ref[...], axis=0) + 1.0
    r = lax.rsqrt(deg)
    row = lax.broadcasted_iota(jnp.int32, (NB, 128), 0)
    col = lax.broadcasted_iota(jnp.int32, (NB, 128), 1)
    dinv_ref[...] = jnp.where(row * 128 + col < N, r, 0.0)


def _mm_body(x_ref, w_ref, dv_ref, o_ref):
    h = jnp.dot(x_ref[...], w_ref[...], preferred_element_type=jnp.float32)
    o_ref[...] = h * dv_ref[...]


def _bn_stats(sums, g, be):
    mu = sums[0:1, :] / N
    var = sums[1:2, :] / N - mu * mu
    scale = g * lax.rsqrt(var + 1e-5)
    shift = be - mu * scale
    return scale, shift


def _acc_pre(p_ref, hs_ref, dv_ref, pre_s, sums_s, i):
    p = jnp.concatenate([p_ref[0], p_ref[1]], axis=1)
    t = (p + hs_ref[...]) * dv_ref[...]
    pre_s[pl.ds(i * 128, 128), :] = t

    @pl.when(i == 0)
    def _():
        sums_s[...] = jnp.zeros_like(sums_s)

    sums_s[0:1, :] += jnp.sum(t, axis=0, keepdims=True)
    sums_s[1:2, :] += jnp.sum(t * t, axis=0, keepdims=True)


def _layer_body(p_ref, hs_ref, dv_ref, g_ref, be_ref, w_ref, o_ref,
                pre_s, sums_s):
    ph = pl.program_id(0)
    i = pl.program_id(1)

    @pl.when(ph == 0)
    def _():
        _acc_pre(p_ref, hs_ref, dv_ref, pre_s, sums_s, i)

    @pl.when(ph == 1)
    def _():
        scale, shift = _bn_stats(sums_s, g_ref[...], be_ref[...])
        y = jnp.maximum(pre_s[pl.ds(i * 128, 128), :] * scale + shift, 0.0)
        h = jnp.dot(y, w_ref[...], preferred_element_type=jnp.float32)
        o_ref[...] = h * dv_ref[...]


def _final_body(p_ref, hs_ref, dv_ref, g_ref, be_ref, w_ref, fb_ref, o_ref,
                pre_s, sums_s):
    ph = pl.program_id(0)
    i = pl.program_id(1)

    @pl.when(ph == 0)
    def _():
        _acc_pre(p_ref, hs_ref, dv_ref, pre_s, sums_s, i)

    @pl.when(ph == 1)
    def _():
        scale, shift = _bn_stats(sums_s, g_ref[...], be_ref[...])
        y = jnp.maximum(pre_s[pl.ds(i * 128, 128), :] * scale + shift, 0.0)
        t = (jnp.dot(y, w_ref[...], preferred_element_type=jnp.float32)
             + fb_ref[...])
        o_ref[...] = jax.nn.sigmoid(t)


_f32 = jnp.float32
_blk = pl.BlockSpec((128, 128), lambda i: (i, 0))
_col = pl.BlockSpec((128, 1), lambda i: (i, 0))
_full = pl.BlockSpec((128, 128), lambda i: (0, 0))
_p2 = pl.BlockSpec((NC, 128, 64), lambda p, i: (0, i * (1 - p), 0))
_blk2 = pl.BlockSpec((128, 128), lambda p, i: (i * (1 - p), 0))
_col2 = pl.BlockSpec((128, 1), lambda p, i: (i, 0))
_full2 = pl.BlockSpec((128, 128), lambda p, i: (0, 0))
_vec2 = pl.BlockSpec((1, 128), lambda p, i: (0, 0))
_out2 = pl.BlockSpec((128, 128), lambda p, i: (p * i, 0))
_scr2 = [pltpu.VMEM((NP, 128), jnp.float32), pltpu.VMEM((8, 128), jnp.float32)]


def _dinv_call(degp):
    return pl.pallas_call(
        _dinv_body,
        out_shape=jax.ShapeDtypeStruct((NB, 128), _f32),
    )(degp.reshape(NT, NB, 128))


def _mm_call(x, w, dv):
    return pl.pallas_call(
        _mm_body, grid=(NB,),
        in_specs=[_blk, _full, _col],
        out_specs=_blk,
        out_shape=jax.ShapeDtypeStruct((NP, 128), _f32),
    )(x, w, dv)


def _layer_call(p, hs, dv, g, be, w):
    return pl.pallas_call(
        _layer_body, grid=(2, NB),
        in_specs=[_p2, _blk2, _col2, _vec2, _vec2, _full2],
        out_specs=_out2,
        out_shape=jax.ShapeDtypeStruct((NP, 128), _f32),
        scratch_shapes=_scr2,
    )(p, hs, dv, g.reshape(1, 128), be.reshape(1, 128), w)


def _final_call(p, hs, dv, g, be, wp, fb):
    return pl.pallas_call(
        _final_body, grid=(2, NB),
        in_specs=[_p2, _blk2, _col2, _vec2, _vec2, _full2, _vec2],
        out_specs=_out2,
        out_shape=jax.ShapeDtypeStruct((NP, 128), _f32),
        scratch_shapes=_scr2,
    )(p, hs, dv, g.reshape(1, 128), be.reshape(1, 128), wp, fb)


def kernel(x, edge_index, W1, b1, g1, be1, W2, b2, g2, be2, W3, b3, g3, be3,
           fcW, fcb):
    del b1, b2, b3
    src = edge_index[0]
    dst = edge_index[1]
    fill = N + jnp.arange(EP - E, dtype=jnp.int32) % (NP - N)
    src_p = jnp.concatenate([src, fill])
    dst_p = jnp.concatenate([dst, fill])
    sb = (2 * src_p).reshape(NS, KB, EPB)
    src3 = jnp.stack([sb, sb + 1])
    dst3 = dst_p.reshape(NS, KB, EPB)
    dst2 = dst_p.reshape(NT, EPD)
    x_pad = jnp.pad(x, ((0, NP - N), (0, 0)))
    zer = jnp.zeros((RPT, 64), _f32)
    fcWp = jnp.pad(fcW, ((0, 0), (0, 127)))
    fbv = jnp.broadcast_to(fcb, (128,)).reshape(1, 128)

    degp = _deg_call(dst2)
    dinv = _dinv_call(degp).reshape(NP, 1)

    hs = _mm_call(x_pad, W1, dinv)
    p1 = _edge_call(hs.reshape(2 * NP, 64), src3, dst3, zer)
    hs2 = _layer_call(p1, hs, dinv, g1, be1, W2)
    p2 = _edge_call(hs2.reshape(2 * NP, 64), src3, dst3, zer)
    hs3 = _layer_call(p2, hs2, dinv, g2, be2, W3)
    p3 = _edge_call(hs3.reshape(2 * NP, 64), src3, dst3, zer)
    res = _final_call(p3, hs3, dinv, g3, be3, fcWp, fbv)
    return res[:N, :1]


# speedup vs baseline: 3.6955x; 1.5622x over previous
import functools

import jax
import jax.numpy as jnp
from jax import lax
from jax.experimental import pallas as pl
from jax.experimental.pallas import tpu as pltpu
from jax.experimental.pallas import tpu_sc as plsc

N = 10000
D = 128
H = 128
E = 320000

NC = 2
NS = 16
NT = NC * NS

NP = 10240
NB = NP // 128
EPB = 128
KB = 160
EPT = KB * EPB
EP = NS * EPT
EPD = EP // NT
RPT = NP // NS

_mesh = plsc.VectorSubcoreMesh(core_axis_name="c", subcore_axis_name="s")
_sc_params = pltpu.CompilerParams(needs_layout_passes=False,
                                 use_tc_tiling_on_sc=False)


@functools.partial(
    pl.kernel,
    out_type=jax.ShapeDtypeStruct((NT, NP), jnp.float32),
    mesh=_mesh,
    compiler_params=_sc_params,
    scratch_types=[
        pltpu.VMEM((EPD,), jnp.int32),
        pltpu.VMEM((NP,), jnp.float32),
    ],
)
def _deg_call(dst_hbm, degp_hbm, dst_v, deg_v):
    c = lax.axis_index("c")
    s = lax.axis_index("s")
    wid = c * NS + s

    def _zero(i, _):
        deg_v[pl.ds(i * 16, 16)] = jnp.zeros((16,), jnp.float32)
        return _

    lax.fori_loop(0, NP // 16, _zero, None)
    pltpu.sync_copy(dst_hbm.at[wid], dst_v)
    ones = jnp.ones((16,), jnp.float32)

    def _count(i, _):
        idx = dst_v[pl.ds(i * 16, 16)]
        plsc.addupdate_scatter(deg_v, [idx], ones)
        return _

    lax.fori_loop(0, EPD // 16, _count, None)
    pltpu.sync_copy(deg_v, degp_hbm.at[wid])


@functools.partial(
    pl.kernel,
    out_type=jax.ShapeDtypeStruct((NC, NP, 64), jnp.float32),
    mesh=_mesh,
    compiler_params=_sc_params,
    scratch_types=[
        pltpu.VMEM((KB, EPB), jnp.int32),
        pltpu.VMEM((KB, EPB), jnp.int32),
        pltpu.VMEM((EPB, 64), jnp.float32),
        pltpu.VMEM((EPB, 64), jnp.float32),
        pltpu.VMEM((EPB, 64), jnp.float32),
        pltpu.VMEM_SHARED((NP, 64), jnp.float32),
        pltpu.SemaphoreType.DMA,
        pltpu.SemaphoreType.DMA,
        pltpu.SemaphoreType.DMA,
    ],
)
def _edge_call(hs_hbm, src_hbm, dst_hbm, zer_hbm, out_hbm,
               src_v, dst_v, rows_a, rows_b, rows_c, acc_sh,
               sem_a, sem_b, sem_c):
    c = lax.axis_index("c")
    s = lax.axis_index("s")

    pltpu.sync_copy(zer_hbm, acc_sh.at[pl.ds(s * RPT, RPT)])
    pltpu.sync_copy(src_hbm.at[c, s], src_v)
    pltpu.sync_copy(dst_hbm.at[s], dst_v)
    plsc.subcore_barrier()

    pltpu.async_copy(hs_hbm.at[src_v.at[0]], rows_a, sem_a)
    pltpu.async_copy(hs_hbm.at[src_v.at[1]], rows_b, sem_b)
    pltpu.async_copy(hs_hbm.at[src_v.at[2]], rows_c, sem_c)

    def _slot(j, buf, sem):
        pltpu.make_async_copy(hs_hbm.at[src_v.at[j]], buf, sem).wait()
        pltpu.sync_copy(buf, acc_sh.at[dst_v.at[j]], add=True)

        @pl.when(j + 3 < KB)
        def _():
            pltpu.async_copy(hs_hbm.at[src_v.at[j + 3]], buf, sem)

    def _triple(t, _):
        j = 3 * t
        _slot(j, rows_a, sem_a)
        _slot(j + 1, rows_b, sem_b)
        _slot(j + 2, rows_c, sem_c)
        return _

    lax.fori_loop(0, KB // 3, _triple, None)
    _slot(KB - 1, rows_a, sem_a)
    plsc.subcore_barrier()
    pltpu.sync_copy(acc_sh.at[pl.ds(s * RPT, RPT)],
                    out_hbm.at[c, pl.ds(s * RPT, RPT)])


def _dinv_body(degp_ref, dinv_ref):
    deg = jnp.sum(degp_ref[...], axis=0) + 1.0
    r = lax.rsqrt(deg)
    row = lax.broadcasted_iota(jnp.int32, (NB, 128), 0)
    col = lax.broadcasted_iota(jnp.int32, (NB, 128), 1)
    dinv_ref[...] = jnp.where(row * 128 + col < N, r, 0.0)


def _mm_body(x_ref, w_ref, dv_ref, o_ref):
    h = jnp.dot(x_ref[...], w_ref[...], preferred_element_type=jnp.float32)
    o_ref[...] = h * dv_ref[...]


def _bn_stats(sums, g, be):
    mu = sums[0:1, :] / N
    var = sums[1:2, :] / N - mu * mu
    scale = g * lax.rsqrt(var + 1e-5)
    shift = be - mu * scale
    return scale, shift


def _acc_pre(p_ref, hs_ref, dv_ref, pre_s, sums_s, i):
    p = jnp.concatenate([p_ref[0], p_ref[1]], axis=1)
    t = (p + hs_ref[...]) * dv_ref[...]
    pre_s[pl.ds(i * BR, BR), :] = t

    @pl.when(i == 0)
    def _():
        sums_s[...] = jnp.zeros_like(sums_s)

    sums_s[0:1, :] += jnp.sum(t, axis=0, keepdims=True)
    sums_s[1:2, :] += jnp.sum(t * t, axis=0, keepdims=True)


def _layer_body(p_ref, hs_ref, dv_ref, g_ref, be_ref, w_ref, o_ref,
                pre_s, sums_s):
    ph = pl.program_id(0)
    i = pl.program_id(1)

    @pl.when(ph == 0)
    def _():
        _acc_pre(p_ref, hs_ref, dv_ref, pre_s, sums_s, i)

    @pl.when(ph == 1)
    def _():
        scale, shift = _bn_stats(sums_s, g_ref[...], be_ref[...])
        y = jnp.maximum(pre_s[pl.ds(i * BR, BR), :] * scale + shift, 0.0)
        h = jnp.dot(y, w_ref[...], preferred_element_type=jnp.float32)
        o_ref[...] = h * dv_ref[...]


def _final_body(p_ref, hs_ref, dv_ref, g_ref, be_ref, w_ref, fb_ref, o_ref,
                pre_s, sums_s):
    ph = pl.program_id(0)
    i = pl.program_id(1)

    @pl.when(ph == 0)
    def _():
        _acc_pre(p_ref, hs_ref, dv_ref, pre_s, sums_s, i)

    @pl.when(ph == 1)
    def _():
        scale, shift = _bn_stats(sums_s, g_ref[...], be_ref[...])
        y = jnp.maximum(pre_s[pl.ds(i * BR, BR), :] * scale + shift, 0.0)
        t = (jnp.dot(y, w_ref[...], preferred_element_type=jnp.float32)
             + fb_ref[...])
        o_ref[...] = jax.nn.sigmoid(t)


_f32 = jnp.float32
BR = 1024
NRB = NP // BR
_blk = pl.BlockSpec((BR, 128), lambda i: (i, 0))
_col = pl.BlockSpec((BR, 1), lambda i: (i, 0))
_full = pl.BlockSpec((128, 128), lambda i: (0, 0))
_p2 = pl.BlockSpec((NC, BR, 64), lambda p, i: (0, i * (1 - p), 0))
_blk2 = pl.BlockSpec((BR, 128), lambda p, i: (i * (1 - p), 0))
_col2 = pl.BlockSpec((BR, 1), lambda p, i: (i, 0))
_full2 = pl.BlockSpec((128, 128), lambda p, i: (0, 0))
_vec2 = pl.BlockSpec((1, 128), lambda p, i: (0, 0))
_out2 = pl.BlockSpec((BR, 128), lambda p, i: (p * i, 0))
_scr2 = [pltpu.VMEM((NP, 128), jnp.float32), pltpu.VMEM((8, 128), jnp.float32)]


def _dinv_call(degp):
    return pl.pallas_call(
        _dinv_body,
        out_shape=jax.ShapeDtypeStruct((NB, 128), _f32),
    )(degp.reshape(NT, NB, 128))


def _mm_call(x, w, dv):
    return pl.pallas_call(
        _mm_body, grid=(NRB,),
        in_specs=[_blk, _full, _col],
        out_specs=_blk,
        out_shape=jax.ShapeDtypeStruct((NP, 128), _f32),
    )(x, w, dv)


def _layer_call(p, hs, dv, g, be, w):
    return pl.pallas_call(
        _layer_body, grid=(2, NRB),
        in_specs=[_p2, _blk2, _col2, _vec2, _vec2, _full2],
        out_specs=_out2,
        out_shape=jax.ShapeDtypeStruct((NP, 128), _f32),
        scratch_shapes=_scr2,
    )(p, hs, dv, g.reshape(1, 128), be.reshape(1, 128), w)


def _final_call(p, hs, dv, g, be, wp, fb):
    return pl.pallas_call(
        _final_body, grid=(2, NRB),
        in_specs=[_p2, _blk2, _col2, _vec2, _vec2, _full2, _vec2],
        out_specs=_out2,
        out_shape=jax.ShapeDtypeStruct((NP, 128), _f32),
        scratch_shapes=_scr2,
    )(p, hs, dv, g.reshape(1, 128), be.reshape(1, 128), wp, fb)


def kernel(x, edge_index, W1, b1, g1, be1, W2, b2, g2, be2, W3, b3, g3, be3,
           fcW, fcb):
    del b1, b2, b3
    src = edge_index[0]
    dst = edge_index[1]
    fill = N + jnp.arange(EP - E, dtype=jnp.int32) % (NP - N)
    src_p = jnp.concatenate([src, fill])
    dst_p = jnp.concatenate([dst, fill])
    sb = (2 * src_p).reshape(NS, KB, EPB)
    src3 = jnp.stack([sb, sb + 1])
    dst3 = dst_p.reshape(NS, KB, EPB)
    dst2 = dst_p.reshape(NT, EPD)
    x_pad = jnp.pad(x, ((0, NP - N), (0, 0)))
    zer = jnp.zeros((RPT, 64), _f32)
    fcWp = jnp.pad(fcW, ((0, 0), (0, 127)))
    fbv = jnp.broadcast_to(fcb, (128,)).reshape(1, 128)

    degp = _deg_call(dst2)
    dinv = _dinv_call(degp).reshape(NP, 1)

    hs = _mm_call(x_pad, W1, dinv)
    p1 = _edge_call(hs.reshape(2 * NP, 64), src3, dst3, zer)
    hs2 = _layer_call(p1, hs, dinv, g1, be1, W2)
    p2 = _edge_call(hs2.reshape(2 * NP, 64), src3, dst3, zer)
    hs3 = _layer_call(p2, hs2, dinv, g2, be2, W3)
    p3 = _edge_call(hs3.reshape(2 * NP, 64), src3, dst3, zer)
    res = _final_call(p3, hs3, dinv, g3, be3, fcWp, fbv)
    return res[:N, :1]


# speedup vs baseline: 3.8468x; 1.0409x over previous
import functools

import jax
import jax.numpy as jnp
from jax import lax
from jax.experimental import pallas as pl
from jax.experimental.pallas import tpu as pltpu
from jax.experimental.pallas import tpu_sc as plsc

N = 10000
D = 128
H = 128
E = 320000

NC = 2
NS = 16
NT = NC * NS

NP = 10240
NB = NP // 128
EPB = 128
KB = 160
EPT = KB * EPB
EP = NS * EPT
EPD = EP // NT
RPT = NP // NS

_mesh = plsc.VectorSubcoreMesh(core_axis_name="c", subcore_axis_name="s")
_sc_params = pltpu.CompilerParams(needs_layout_passes=False,
                                 use_tc_tiling_on_sc=False)


@functools.partial(
    pl.kernel,
    out_type=jax.ShapeDtypeStruct((NT, NP), jnp.float32),
    mesh=_mesh,
    compiler_params=_sc_params,
    scratch_types=[
        pltpu.VMEM((EPD,), jnp.int32),
        pltpu.VMEM((NP,), jnp.float32),
    ],
)
def _deg_call(dst_hbm, degp_hbm, dst_v, deg_v):
    c = lax.axis_index("c")
    s = lax.axis_index("s")
    wid = c * NS + s

    def _zero(i, _):
        deg_v[pl.ds(i * 16, 16)] = jnp.zeros((16,), jnp.float32)
        return _

    lax.fori_loop(0, NP // 16, _zero, None)
    pltpu.sync_copy(dst_hbm.at[wid], dst_v)
    ones = jnp.ones((16,), jnp.float32)

    def _count(i, _):
        idx = dst_v[pl.ds(i * 16, 16)]
        plsc.addupdate_scatter(deg_v, [idx], ones)
        return _

    lax.fori_loop(0, EPD // 16, _count, None)
    pltpu.sync_copy(deg_v, degp_hbm.at[wid])


@functools.partial(
    pl.kernel,
    out_type=jax.ShapeDtypeStruct((NC, NP, 64), jnp.float32),
    mesh=_mesh,
    compiler_params=_sc_params,
    scratch_types=[
        pltpu.VMEM((KB, EPB), jnp.int32),
        pltpu.VMEM((KB, EPB), jnp.int32),
        pltpu.VMEM((EPB, 64), jnp.float32),
        pltpu.VMEM((EPB, 64), jnp.float32),
        pltpu.VMEM((EPB, 64), jnp.float32),
        pltpu.VMEM_SHARED((NP, 64), jnp.float32),
        pltpu.SemaphoreType.DMA,
        pltpu.SemaphoreType.DMA,
        pltpu.SemaphoreType.DMA,
    ],
)
def _edge_call(hs_hbm, src_hbm, dst_hbm, zer_hbm, out_hbm,
               src_v, dst_v, rows_a, rows_b, rows_c, acc_sh,
               sem_a, sem_b, sem_c):
    c = lax.axis_index("c")
    s = lax.axis_index("s")

    pltpu.sync_copy(zer_hbm, acc_sh.at[pl.ds(s * RPT, RPT)])
    pltpu.sync_copy(src_hbm.at[c, s], src_v)
    pltpu.sync_copy(dst_hbm.at[s], dst_v)
    plsc.subcore_barrier()

    pltpu.async_copy(hs_hbm.at[src_v.at[0]], rows_a, sem_a)
    pltpu.async_copy(hs_hbm.at[src_v.at[1]], rows_b, sem_b)
    pltpu.async_copy(hs_hbm.at[src_v.at[2]], rows_c, sem_c)

    def _slot(j, buf, sem):
        pltpu.make_async_copy(hs_hbm.at[src_v.at[j]], buf, sem).wait()
        pltpu.sync_copy(buf, acc_sh.at[dst_v.at[j]], add=True)

        @pl.when(j + 3 < KB)
        def _():
            pltpu.async_copy(hs_hbm.at[src_v.at[j + 3]], buf, sem)

    def _triple(t, _):
        j = 3 * t
        _slot(j, rows_a, sem_a)
        _slot(j + 1, rows_b, sem_b)
        _slot(j + 2, rows_c, sem_c)
        return _

    lax.fori_loop(0, KB // 3, _triple, None)
    _slot(KB - 1, rows_a, sem_a)
    plsc.subcore_barrier()
    pltpu.sync_copy(acc_sh.at[pl.ds(s * RPT, RPT)],
                    out_hbm.at[c, pl.ds(s * RPT, RPT)])


def _dinv_body(degp_ref, dinv_ref):
    deg = jnp.sum(degp_ref[...], axis=0) + 1.0
    r = lax.rsqrt(deg)
    row = lax.broadcasted_iota(jnp.int32, (NB, 128), 0)
    col = lax.broadcasted_iota(jnp.int32, (NB, 128), 1)
    dinv_ref[...] = jnp.where(row * 128 + col < N, r, 0.0)


def _mm_body(x_ref, w_ref, dv_ref, o_ref):
    h = jnp.dot(x_ref[...], w_ref[...], preferred_element_type=jnp.float32)
    o_ref[...] = h * dv_ref[...]


def _bn_stats(sums, g, be):
    mu = sums[0:1, :] / N
    var = sums[1:2, :] / N - mu * mu
    scale = g * lax.rsqrt(var + 1e-5)
    shift = be - mu * scale
    return scale, shift


def _acc_pre(p_ref, hs_ref, dv_ref, pre_s, sums_s, i):
    p = jnp.concatenate([p_ref[0], p_ref[1]], axis=1)
    t = (p + hs_ref[...]) * dv_ref[...]
    pre_s[pl.ds(i * BR, BR), :] = t

    @pl.when(i == 0)
    def _():
        sums_s[...] = jnp.zeros_like(sums_s)

    sums_s[0:1, :] += jnp.sum(t, axis=0, keepdims=True)
    sums_s[1:2, :] += jnp.sum(t * t, axis=0, keepdims=True)


def _layer_body(p_ref, hs_ref, dv_ref, g_ref, be_ref, w_ref, o_ref,
                pre_s, sums_s):
    ph = pl.program_id(0)
    i = pl.program_id(1)

    @pl.when(ph == 0)
    def _():
        _acc_pre(p_ref, hs_ref, dv_ref, pre_s, sums_s, i)

    @pl.when(ph == 1)
    def _():
        scale, shift = _bn_stats(sums_s, g_ref[...], be_ref[...])
        y = jnp.maximum(pre_s[pl.ds(i * BR, BR), :] * scale + shift, 0.0)
        h = jnp.dot(y, w_ref[...], preferred_element_type=jnp.float32)
        o_ref[...] = h * dv_ref[...]


def _final_body(p_ref, hs_ref, dv_ref, g_ref, be_ref, w_ref, fb_ref, o_ref,
                pre_s, sums_s):
    ph = pl.program_id(0)
    i = pl.program_id(1)

    @pl.when(ph == 0)
    def _():
        _acc_pre(p_ref, hs_ref, dv_ref, pre_s, sums_s, i)

    @pl.when(ph == 1)
    def _():
        scale, shift = _bn_stats(sums_s, g_ref[...], be_ref[...])
        y = jnp.maximum(pre_s[pl.ds(i * BR, BR), :] * scale + shift, 0.0)
        t = (jnp.dot(y, w_ref[...], preferred_element_type=jnp.float32)
             + fb_ref[...])
        o_ref[...] = jax.nn.sigmoid(t)


_f32 = jnp.float32
BR = 2048
NRB = NP // BR
_blk = pl.BlockSpec((BR, 128), lambda i: (i, 0))
_col = pl.BlockSpec((BR, 1), lambda i: (i, 0))
_full = pl.BlockSpec((128, 128), lambda i: (0, 0))
_p2 = pl.BlockSpec((NC, BR, 64), lambda p, i: (0, i * (1 - p), 0))
_blk2 = pl.BlockSpec((BR, 128), lambda p, i: (i * (1 - p), 0))
_col2 = pl.BlockSpec((BR, 1), lambda p, i: (i, 0))
_full2 = pl.BlockSpec((128, 128), lambda p, i: (0, 0))
_vec2 = pl.BlockSpec((1, 128), lambda p, i: (0, 0))
_out2 = pl.BlockSpec((BR, 128), lambda p, i: (p * i, 0))
_scr2 = [pltpu.VMEM((NP, 128), jnp.float32), pltpu.VMEM((8, 128), jnp.float32)]


def _dinv_call(degp):
    return pl.pallas_call(
        _dinv_body,
        out_shape=jax.ShapeDtypeStruct((NB, 128), _f32),
    )(degp.reshape(NT, NB, 128))


def _mm_call(x, w, dv):
    return pl.pallas_call(
        _mm_body, grid=(NRB,),
        in_specs=[_blk, _full, _col],
        out_specs=_blk,
        out_shape=jax.ShapeDtypeStruct((NP, 128), _f32),
    )(x, w, dv)


def _layer_call(p, hs, dv, g, be, w):
    return pl.pallas_call(
        _layer_body, grid=(2, NRB),
        in_specs=[_p2, _blk2, _col2, _vec2, _vec2, _full2],
        out_specs=_out2,
        out_shape=jax.ShapeDtypeStruct((NP, 128), _f32),
        scratch_shapes=_scr2,
    )(p, hs, dv, g.reshape(1, 128), be.reshape(1, 128), w)


def _final_call(p, hs, dv, g, be, wp, fb):
    return pl.pallas_call(
        _final_body, grid=(2, NRB),
        in_specs=[_p2, _blk2, _col2, _vec2, _vec2, _full2, _vec2],
        out_specs=_out2,
        out_shape=jax.ShapeDtypeStruct((NP, 128), _f32),
        scratch_shapes=_scr2,
    )(p, hs, dv, g.reshape(1, 128), be.reshape(1, 128), wp, fb)


def kernel(x, edge_index, W1, b1, g1, be1, W2, b2, g2, be2, W3, b3, g3, be3,
           fcW, fcb):
    del b1, b2, b3
    src = edge_index[0]
    dst = edge_index[1]
    fill = N + jnp.arange(EP - E, dtype=jnp.int32) % (NP - N)
    src_p = jnp.concatenate([src, fill])
    dst_p = jnp.concatenate([dst, fill])
    sb = (2 * src_p).reshape(NS, KB, EPB)
    src3 = jnp.stack([sb, sb + 1])
    dst3 = dst_p.reshape(NS, KB, EPB)
    dst2 = dst_p.reshape(NT, EPD)
    x_pad = jnp.pad(x, ((0, NP - N), (0, 0)))
    zer = jnp.zeros((RPT, 64), _f32)
    fcWp = jnp.pad(fcW, ((0, 0), (0, 127)))
    fbv = jnp.broadcast_to(fcb, (128,)).reshape(1, 128)

    degp = _deg_call(dst2)
    dinv = _dinv_call(degp).reshape(NP, 1)

    hs = _mm_call(x_pad, W1, dinv)
    p1 = _edge_call(hs.reshape(2 * NP, 64), src3, dst3, zer)
    hs2 = _layer_call(p1, hs, dinv, g1, be1, W2)
    p2 = _edge_call(hs2.reshape(2 * NP, 64), src3, dst3, zer)
    hs3 = _layer_call(p2, hs2, dinv, g2, be2, W3)
    p3 = _edge_call(hs3.reshape(2 * NP, 64), src3, dst3, zer)
    res = _final_call(p3, hs3, dinv, g3, be3, fcWp, fbv)
    return res[:N, :1]


# speedup vs baseline: 3.8533x; 1.0017x over previous
import functools

import jax
import jax.numpy as jnp
from jax import lax
from jax.experimental import pallas as pl
from jax.experimental.pallas import tpu as pltpu
from jax.experimental.pallas import tpu_sc as plsc

N = 10000
D = 128
H = 128
E = 320000

NC = 2
NS = 16
NT = NC * NS

NP = 10240
NB = NP // 128
EPB = 128
KB = 160
EPT = KB * EPB
EP = NS * EPT
EPD = EP // NT
RPT = NP // NS

_mesh = plsc.VectorSubcoreMesh(core_axis_name="c", subcore_axis_name="s")
_sc_params = pltpu.CompilerParams(needs_layout_passes=False,
                                 use_tc_tiling_on_sc=False)


@functools.partial(
    pl.kernel,
    out_type=jax.ShapeDtypeStruct((NT, NP), jnp.float32),
    mesh=_mesh,
    compiler_params=_sc_params,
    scratch_types=[
        pltpu.VMEM((EPD,), jnp.int32),
        pltpu.VMEM((NP,), jnp.float32),
    ],
)
def _deg_call(dst_hbm, degp_hbm, dst_v, deg_v):
    c = lax.axis_index("c")
    s = lax.axis_index("s")
    wid = c * NS + s

    def _zero(i, _):
        deg_v[pl.ds(i * 16, 16)] = jnp.zeros((16,), jnp.float32)
        return _

    lax.fori_loop(0, NP // 16, _zero, None)
    pltpu.sync_copy(dst_hbm.at[wid], dst_v)
    ones = jnp.ones((16,), jnp.float32)

    def _count(i, _):
        idx = dst_v[pl.ds(i * 16, 16)]
        plsc.addupdate_scatter(deg_v, [idx], ones)
        return _

    lax.fori_loop(0, EPD // 16, _count, None)
    pltpu.sync_copy(deg_v, degp_hbm.at[wid])


@functools.partial(
    pl.kernel,
    out_type=jax.ShapeDtypeStruct((NC, NP, 64), jnp.float32),
    mesh=_mesh,
    compiler_params=_sc_params,
    scratch_types=[
        pltpu.VMEM((KB, EPB), jnp.int32),
        pltpu.VMEM((KB, EPB), jnp.int32),
        pltpu.VMEM((EPB, 64), jnp.float32),
        pltpu.VMEM((EPB, 64), jnp.float32),
        pltpu.VMEM((EPB, 64), jnp.float32),
        pltpu.VMEM_SHARED((NP, 64), jnp.float32),
        pltpu.SemaphoreType.DMA,
        pltpu.SemaphoreType.DMA,
        pltpu.SemaphoreType.DMA,
    ],
)
def _edge_call(hs_hbm, src_hbm, dst_hbm, zer_hbm, out_hbm,
               src_v, dst_v, rows_a, rows_b, rows_c, acc_sh,
               sem_a, sem_b, sem_c):
    c = lax.axis_index("c")
    s = lax.axis_index("s")

    pltpu.sync_copy(zer_hbm, acc_sh.at[pl.ds(s * RPT, RPT)])
    pltpu.sync_copy(src_hbm.at[c, s], src_v)
    pltpu.sync_copy(dst_hbm.at[s], dst_v)
    plsc.subcore_barrier()

    pltpu.async_copy(hs_hbm.at[src_v.at[0]], rows_a, sem_a)
    pltpu.async_copy(hs_hbm.at[src_v.at[1]], rows_b, sem_b)
    pltpu.async_copy(hs_hbm.at[src_v.at[2]], rows_c, sem_c)

    def _slot(j, buf, sem):
        pltpu.make_async_copy(hs_hbm.at[src_v.at[j]], buf, sem).wait()
        pltpu.sync_copy(buf, acc_sh.at[dst_v.at[j]], add=True)

        @pl.when(j + 3 < KB)
        def _():
            pltpu.async_copy(hs_hbm.at[src_v.at[j + 3]], buf, sem)

    def _triple(t, _):
        j = 3 * t
        _slot(j, rows_a, sem_a)
        _slot(j + 1, rows_b, sem_b)
        _slot(j + 2, rows_c, sem_c)
        return _

    lax.fori_loop(0, KB // 3, _triple, None)
    _slot(KB - 1, rows_a, sem_a)
    plsc.subcore_barrier()
    pltpu.sync_copy(acc_sh.at[pl.ds(s * RPT, RPT)],
                    out_hbm.at[c, pl.ds(s * RPT, RPT)])


def _dinv_body(degp_ref, dinv_ref):
    deg = jnp.sum(degp_ref[...], axis=0) + 1.0
    r = lax.rsqrt(deg)
    row = lax.broadcasted_iota(jnp.int32, (NB, 128), 0)
    col = lax.broadcasted_iota(jnp.int32, (NB, 128), 1)
    dinv_ref[...] = jnp.where(row * 128 + col < N, r, 0.0)


def _mm_body(x_ref, w_ref, dv_ref, o_ref):
    h = jnp.dot(x_ref[...], w_ref[...], preferred_element_type=jnp.float32)
    row = (lax.broadcasted_iota(jnp.int32, (BR, 1), 0)
           + pl.program_id(0) * BR)
    o_ref[...] = jnp.where(row < N, h * dv_ref[...], 0.0)


def _bn_stats(sums, g, be):
    mu = sums[0:1, :] / N
    var = sums[1:2, :] / N - mu * mu
    scale = g * lax.rsqrt(var + 1e-5)
    shift = be - mu * scale
    return scale, shift


def _acc_pre(p_ref, hs_ref, dv_ref, pre_s, sums_s, i):
    p = jnp.concatenate([p_ref[0], p_ref[1]], axis=1)
    t = (p + hs_ref[...]) * dv_ref[...]
    pre_s[pl.ds(i * BR, BR), :] = t

    @pl.when(i == 0)
    def _():
        sums_s[...] = jnp.zeros_like(sums_s)

    sums_s[0:1, :] += jnp.sum(t, axis=0, keepdims=True)
    sums_s[1:2, :] += jnp.sum(t * t, axis=0, keepdims=True)


def _layer_body(p_ref, hs_ref, dv_ref, g_ref, be_ref, w_ref, o_ref,
                pre_s, sums_s):
    ph = pl.program_id(0)
    i = pl.program_id(1)

    @pl.when(ph == 0)
    def _():
        _acc_pre(p_ref, hs_ref, dv_ref, pre_s, sums_s, i)

    @pl.when(ph == 1)
    def _():
        scale, shift = _bn_stats(sums_s, g_ref[...], be_ref[...])
        y = jnp.maximum(pre_s[pl.ds(i * BR, BR), :] * scale + shift, 0.0)
        h = jnp.dot(y, w_ref[...], preferred_element_type=jnp.float32)
        o_ref[...] = h * dv_ref[...]


def _final_body(p_ref, hs_ref, dv_ref, g_ref, be_ref, w_ref, fb_ref, o_ref,
                pre_s, sums_s):
    ph = pl.program_id(0)
    i = pl.program_id(1)

    @pl.when(ph == 0)
    def _():
        _acc_pre(p_ref, hs_ref, dv_ref, pre_s, sums_s, i)

    @pl.when(ph == 1)
    def _():
        scale, shift = _bn_stats(sums_s, g_ref[...], be_ref[...])
        y = jnp.maximum(pre_s[pl.ds(i * BR, BR), :] * scale + shift, 0.0)
        t = (jnp.dot(y, w_ref[...], preferred_element_type=jnp.float32)
             + fb_ref[...])
        o_ref[...] = jax.nn.sigmoid(t)


_f32 = jnp.float32
BR = 2048
NRB = NP // BR
_blk = pl.BlockSpec((BR, 128), lambda i: (i, 0))
_col = pl.BlockSpec((BR, 1), lambda i: (i, 0))
_full = pl.BlockSpec((128, 128), lambda i: (0, 0))
_p2 = pl.BlockSpec((NC, BR, 64), lambda p, i: (0, i * (1 - p), 0))
_blk2 = pl.BlockSpec((BR, 128), lambda p, i: (i * (1 - p), 0))
_col2 = pl.BlockSpec((BR, 1), lambda p, i: (i, 0))
_full2 = pl.BlockSpec((128, 128), lambda p, i: (0, 0))
_vec2 = pl.BlockSpec((1, 128), lambda p, i: (0, 0))
_out2 = pl.BlockSpec((BR, 128), lambda p, i: (p * i, 0))
_scr2 = [pltpu.VMEM((NP, 128), jnp.float32), pltpu.VMEM((8, 128), jnp.float32)]


def _dinv_call(degp):
    return pl.pallas_call(
        _dinv_body,
        out_shape=jax.ShapeDtypeStruct((NB, 128), _f32),
    )(degp.reshape(NT, NB, 128))


def _mm_call(x, w, dv):
    return pl.pallas_call(
        _mm_body, grid=(NRB,),
        in_specs=[_blk, _full, _col],
        out_specs=_blk,
        out_shape=jax.ShapeDtypeStruct((NP, 128), _f32),
    )(x, w, dv)


def _layer_call(p, hs, dv, g, be, w):
    return pl.pallas_call(
        _layer_body, grid=(2, NRB),
        in_specs=[_p2, _blk2, _col2, _vec2, _vec2, _full2],
        out_specs=_out2,
        out_shape=jax.ShapeDtypeStruct((NP, 128), _f32),
        scratch_shapes=_scr2,
    )(p, hs, dv, g.reshape(1, 128), be.reshape(1, 128), w)


def _final_call(p, hs, dv, g, be, wp, fb):
    return pl.pallas_call(
        _final_body, grid=(2, NRB),
        in_specs=[_p2, _blk2, _col2, _vec2, _vec2, _full2, _vec2],
        out_specs=_out2,
        out_shape=jax.ShapeDtypeStruct((NP, 128), _f32),
        scratch_shapes=_scr2,
    )(p, hs, dv, g.reshape(1, 128), be.reshape(1, 128), wp, fb)


def kernel(x, edge_index, W1, b1, g1, be1, W2, b2, g2, be2, W3, b3, g3, be3,
           fcW, fcb):
    del b1, b2, b3
    src = edge_index[0]
    dst = edge_index[1]
    fill = N + jnp.arange(EP - E, dtype=jnp.int32) % (NP - N)
    src_p = jnp.concatenate([src, fill])
    dst_p = jnp.concatenate([dst, fill])
    sb = (2 * src_p).reshape(NS, KB, EPB)
    src3 = jnp.stack([sb, sb + 1])
    dst3 = dst_p.reshape(NS, KB, EPB)
    dst2 = dst_p.reshape(NT, EPD)
    zer = jnp.zeros((RPT, 64), _f32)
    fcWp = jnp.pad(fcW, ((0, 0), (0, 127)))
    fbv = jnp.broadcast_to(fcb, (128,)).reshape(1, 128)

    degp = _deg_call(dst2)
    dinv = _dinv_call(degp).reshape(NP, 1)

    hs = _mm_call(x, W1, dinv)
    p1 = _edge_call(hs.reshape(2 * NP, 64), src3, dst3, zer)
    hs2 = _layer_call(p1, hs, dinv, g1, be1, W2)
    p2 = _edge_call(hs2.reshape(2 * NP, 64), src3, dst3, zer)
    hs3 = _layer_call(p2, hs2, dinv, g2, be2, W3)
    p3 = _edge_call(hs3.reshape(2 * NP, 64), src3, dst3, zer)
    res = _final_call(p3, hs3, dinv, g3, be3, fcWp, fbv)
    return res[:N, :1]
